# baseline (speedup 1.0000x reference)
import functools

import jax
import jax.numpy as jnp
from jax import lax
from jax.experimental import pallas as pl
from jax.experimental.pallas import tpu as pltpu
from jax.experimental.pallas import tpu_sc as plsc

B = 1024
CAST = 1000
IN = 64
H = 64

_NC = 2
_NS = 16
_L = 16
_NW = _NC * _NS
_BPW = B // _NW
_NCHUNK = _BPW * H // 128


@functools.lru_cache(maxsize=None)
def _make_sc_gather():
    @functools.partial(
        pl.kernel,
        mesh=plsc.VectorSubcoreMesh(core_axis_name="c", subcore_axis_name="s"),
        out_type=jax.ShapeDtypeStruct((_NW, _NCHUNK, 128), jnp.float32),
        scratch_types=[
            pltpu.VMEM((_BPW,), jnp.int32),
            pltpu.VMEM((_NCHUNK, 128), jnp.int32),
            pltpu.VMEM((_NCHUNK, 128), jnp.float32),
            pltpu.SemaphoreType.DMA,
        ],
        compiler_params=pltpu.CompilerParams(needs_layout_passes=False,
                                             skip_device_barrier=True),
    )
    def _sc_gather(ids_hbm, flat_hbm, out_hbm, ids_v, idx_v, rows_v, sem):
        wid = lax.axis_index("s") * _NC + lax.axis_index("c")
        base = wid * _BPW
        pltpu.sync_copy(ids_hbm.at[pl.ds(base, _BPW)], ids_v)
        lanes = lax.iota(jnp.int32, _L)
        av = []
        for half in range(_BPW // _L):
            a = jnp.clip(ids_v[pl.ds(half * _L, _L)], 0, CAST - 1)
            i = base + half * _L + lanes
            av.append(a * (H * 1024) + (i // 128) * 1024 + (i % 128))
        descs = []
        for q in range(_NCHUNK):
            for gg in range(8):
                g = q * 8 + gg
                h = g // (_BPW // _L)
                half = g % (_BPW // _L)
                hoff = (h // 8) * 8192 + (h % 8) * 128
                idx_v[q, pl.ds(gg * _L, _L)] = av[half] + hoff
            descs.append(pltpu.async_copy(flat_hbm.at[idx_v.at[q]],
                                          rows_v.at[q], sem))
        for d in descs:
            d.wait()
        pltpu.sync_copy(rows_v, out_hbm.at[wid])

    return _sc_gather


def _gi_body(xt_ref, wih_t, gi_ref):
    gi_ref[...] = lax.dot_general(wih_t[...], xt_ref[...],
                                  (((0,), (0,)), ((), ())),
                                  preferred_element_type=jnp.float32)


_gi = pl.pallas_call(
    _gi_body,
    out_shape=jax.ShapeDtypeStruct((3 * H, B), jnp.float32),
)


def _gru_body(gi_ref, ht_ref, whh_t, br, bz, bin_, bhn, ot_ref):
    gi = gi_ref[...]
    ht = ht_ref[...]
    gh = lax.dot_general(whh_t[...], ht, (((0,), (0,)), ((), ())),
                         preferred_element_type=jnp.float32)
    r = jax.nn.sigmoid(gi[0:H] + gh[0:H] + br[...])
    z = jax.nn.sigmoid(gi[H:2 * H] + gh[H:2 * H] + bz[...])
    hn = gh[2 * H:] + bhn[...]
    n = jnp.tanh(gi[2 * H:] + bin_[...] + r * hn)
    ot_ref[...] = (1.0 - z) * n + z * ht


_gru = pl.pallas_call(
    _gru_body,
    out_shape=jax.ShapeDtypeStruct((H, B), jnp.float32),
)


def kernel(x, batch_idxs, actor_ids, story_stop_idxs, state, W_ih, W_hh, b_ih, b_hh):
    st_t = jnp.transpose(state, (1, 2, 0))
    st_5d = st_t.reshape(CAST, 8, H // 8, 8, 128)
    st_phys = jnp.transpose(st_5d, (0, 1, 3, 2, 4))
    flat = st_phys.reshape(-1)
    g = _make_sc_gather()(actor_ids, flat)
    ht = g.reshape(_NW, H, _BPW).transpose(1, 0, 2).reshape(H, B)
    xt = x.T
    br = (b_ih[:H] + b_hh[:H]).reshape(H, 1)
    bz = (b_ih[H:2 * H] + b_hh[H:2 * H]).reshape(H, 1)
    bin_ = b_ih[2 * H:].reshape(H, 1)
    bhn = b_hh[2 * H:].reshape(H, 1)
    gi = _gi(xt, W_ih.T)
    out_t = _gru(gi, ht, W_hh.T, br, bz, bin_, bhn)
    return out_t.T


# speedup vs baseline: 1.0163x; 1.0163x over previous
import functools

import jax
import jax.numpy as jnp
from jax import lax
from jax.experimental import pallas as pl
from jax.experimental.pallas import tpu as pltpu
from jax.experimental.pallas import tpu_sc as plsc

B = 1024
CAST = 1000
IN = 64
H = 64

_NC = 2
_NS = 16
_L = 16
_NW = _NC * _NS
_BPW = B // _NW
_NCHUNK = _BPW * H // 128


@functools.lru_cache(maxsize=None)
def _make_sc_gather():
    @functools.partial(
        pl.kernel,
        mesh=plsc.VectorSubcoreMesh(core_axis_name="c", subcore_axis_name="s"),
        out_type=jax.ShapeDtypeStruct((_NW, _NCHUNK, 128), jnp.float32),
        scratch_types=[
            pltpu.VMEM((_BPW,), jnp.int32),
            pltpu.VMEM((_NCHUNK, 128), jnp.int32),
            pltpu.VMEM((_NCHUNK, 128), jnp.float32),
            pltpu.SemaphoreType.DMA,
        ],
        compiler_params=pltpu.CompilerParams(needs_layout_passes=False,
                                             skip_device_barrier=True),
    )
    def _sc_gather(ids_hbm, flat_hbm, out_hbm, ids_v, idx_v, rows_v, sem):
        wid = lax.axis_index("s") * _NC + lax.axis_index("c")
        base = wid * _BPW
        pltpu.sync_copy(ids_hbm.at[pl.ds(base, _BPW)], ids_v)
        lanes = lax.iota(jnp.int32, _L)
        av = []
        for half in range(_BPW // _L):
            a = jnp.clip(ids_v[pl.ds(half * _L, _L)], 0, CAST - 1)
            i = base + half * _L + lanes
            av.append(a * (H * 1024) + (i // 128) * 1024 + (i % 128))
        descs = []
        for q in range(_NCHUNK):
            for gg in range(8):
                g = q * 8 + gg
                h = g // (_BPW // _L)
                half = g % (_BPW // _L)
                hoff = (h // 8) * 8192 + (h % 8) * 128
                idx_v[q, pl.ds(gg * _L, _L)] = av[half] + hoff
            descs.append(pltpu.async_copy(flat_hbm.at[idx_v.at[q]],
                                          rows_v.at[q], sem))
        for d in descs:
            d.wait()
        pltpu.sync_copy(rows_v, out_hbm.at[wid])

    return _sc_gather


def _gru_body(xt_ref, ht_ref, wih_t, whh_t, br, bz, bin_, bhn, ot_ref):
    ht = ht_ref[...]
    gi = lax.dot_general(wih_t[...], xt_ref[...], (((0,), (0,)), ((), ())),
                         preferred_element_type=jnp.float32)
    gh = lax.dot_general(whh_t[...], ht, (((0,), (0,)), ((), ())),
                         preferred_element_type=jnp.float32)
    r = jax.nn.sigmoid(gi[0:H] + gh[0:H] + br[...])
    z = jax.nn.sigmoid(gi[H:2 * H] + gh[H:2 * H] + bz[...])
    hn = gh[2 * H:] + bhn[...]
    n = jnp.tanh(gi[2 * H:] + bin_[...] + r * hn)
    ot_ref[...] = (1.0 - z) * n + z * ht


_gru = pl.pallas_call(
    _gru_body,
    out_shape=jax.ShapeDtypeStruct((H, B), jnp.float32),
)


def kernel(x, batch_idxs, actor_ids, story_stop_idxs, state, W_ih, W_hh, b_ih, b_hh):
    st_t = jnp.transpose(state, (1, 2, 0))
    st_5d = st_t.reshape(CAST, 8, H // 8, 8, 128)
    st_phys = jnp.transpose(st_5d, (0, 1, 3, 2, 4))
    flat = st_phys.reshape(-1)
    g = _make_sc_gather()(actor_ids, flat)
    ht = g.reshape(_NW, H, _BPW).transpose(1, 0, 2).reshape(H, B)
    xt = x.T
    br = (b_ih[:H] + b_hh[:H]).reshape(H, 1)
    bz = (b_ih[H:2 * H] + b_hh[H:2 * H]).reshape(H, 1)
    bin_ = b_ih[2 * H:].reshape(H, 1)
    bhn = b_hh[2 * H:].reshape(H, 1)
    out_t = _gru(xt, ht, W_ih.T, W_hh.T, br, bz, bin_, bhn)
    return out_t.T
